# baseline (speedup 1.0000x reference)
import functools

import jax
import jax.numpy as jnp
from jax.experimental import pallas as pl

N_LAYERS = 4
K = 1024
DIM = 64
TILE = 512


def _body(x_ref, cb_ref, out_ref):
    resid = x_ref[...]
    codes = []
    for l in range(N_LAYERS):
        cb = cb_ref[l]
        cb_norm = jnp.sum(cb * cb, axis=1)[None, :]
        x_norm = jnp.sum(resid * resid, axis=1, keepdims=True)
        mm = jax.lax.dot_general(
            resid, cb, (((1,), (1,)), ((), ())),
            preferred_element_type=jnp.float32,
        )
        d = (x_norm + cb_norm) - 2.0 * mm
        d_min = jnp.min(d, axis=1, keepdims=True)
        iota = jax.lax.broadcasted_iota(jnp.int32, d.shape, 1)
        code = jnp.min(jnp.where(d == d_min, iota, K), axis=1, keepdims=True)
        onehot = (iota == code).astype(jnp.float32)
        delta = jax.lax.dot_general(
            onehot, cb, (((1,), (0,)), ((), ())),
            precision=jax.lax.Precision.HIGHEST,
            preferred_element_type=jnp.float32,
        )
        resid = resid - delta
        codes.append(code)
    out_ref[...] = jnp.concatenate(codes, axis=1)


@jax.jit
def kernel(x, centroids):
    n = x.shape[0]
    return pl.pallas_call(
        _body,
        grid=(n // TILE,),
        in_specs=[
            pl.BlockSpec((TILE, DIM), lambda i: (i, 0)),
            pl.BlockSpec((N_LAYERS, K, DIM), lambda i: (0, 0, 0)),
        ],
        out_specs=pl.BlockSpec((TILE, N_LAYERS), lambda i: (i, 0)),
        out_shape=jax.ShapeDtypeStruct((n, N_LAYERS), jnp.int32),
    )(x, centroids)


# speedup vs baseline: 1.4271x; 1.4271x over previous
import functools

import jax
import jax.numpy as jnp
from jax.experimental import pallas as pl

N_LAYERS = 4
K = 1024
DIM = 64
TILE = 512


def _body(x_ref, cb_ref, cbn_ref, hi_ref, mid_ref, lo_ref, out_ref):
    resid = x_ref[...]
    codes = []
    for l in range(N_LAYERS):
        cb_norm = cbn_ref[l][None, :]
        x_norm = jnp.sum(resid * resid, axis=1, keepdims=True)
        mm = jax.lax.dot_general(
            resid, cb_ref[l], (((1,), (1,)), ((), ())),
            preferred_element_type=jnp.float32,
        )
        d = (x_norm + cb_norm) - 2.0 * mm
        d_min = jnp.min(d, axis=1, keepdims=True)
        iota = jax.lax.broadcasted_iota(jnp.int32, d.shape, 1)
        code = jnp.min(jnp.where(d == d_min, iota, K), axis=1, keepdims=True)
        onehot = (iota == code).astype(jnp.float32).astype(jnp.bfloat16)
        dn = (((1,), (0,)), ((), ()))
        delta = (
            jax.lax.dot_general(onehot, hi_ref[l], dn,
                                preferred_element_type=jnp.float32)
            + jax.lax.dot_general(onehot, mid_ref[l], dn,
                                  preferred_element_type=jnp.float32)
            + jax.lax.dot_general(onehot, lo_ref[l], dn,
                                  preferred_element_type=jnp.float32)
        )
        resid = resid - delta
        codes.append(code)
    out_ref[...] = jnp.concatenate(codes, axis=1)


@jax.jit
def kernel(x, centroids):
    n = x.shape[0]
    cb_norm = jnp.sum(centroids * centroids, axis=2)
    hi = centroids.astype(jnp.bfloat16)
    r1 = centroids - hi.astype(jnp.float32)
    mid = r1.astype(jnp.bfloat16)
    lo = (r1 - mid.astype(jnp.float32)).astype(jnp.bfloat16)
    full = lambda s: pl.BlockSpec(s, lambda i: (0,) * len(s))
    return pl.pallas_call(
        _body,
        grid=(n // TILE,),
        in_specs=[
            pl.BlockSpec((TILE, DIM), lambda i: (i, 0)),
            full((N_LAYERS, K, DIM)),
            full((N_LAYERS, K)),
            full((N_LAYERS, K, DIM)),
            full((N_LAYERS, K, DIM)),
            full((N_LAYERS, K, DIM)),
        ],
        out_specs=pl.BlockSpec((TILE, N_LAYERS), lambda i: (i, 0)),
        out_shape=jax.ShapeDtypeStruct((n, N_LAYERS), jnp.int32),
    )(x, centroids, cb_norm, hi, mid, lo)


# speedup vs baseline: 1.8290x; 1.2817x over previous
import functools

import jax
import jax.numpy as jnp
from jax.experimental import pallas as pl

N_LAYERS = 4
K = 1024
DIM = 64
HALF = 256
TILE = 2 * HALF


def _layer(resid, cb, cb_norm, hi, mid, lo):
    x_norm = jnp.sum(resid * resid, axis=1, keepdims=True)
    mm = jax.lax.dot_general(
        resid, cb, (((1,), (1,)), ((), ())),
        preferred_element_type=jnp.float32,
    )
    d = (x_norm + cb_norm) - 2.0 * mm
    d_min = jnp.min(d, axis=1, keepdims=True)
    iota = jax.lax.broadcasted_iota(jnp.int32, d.shape, 1)
    code = jnp.min(jnp.where(d == d_min, iota, K), axis=1, keepdims=True)
    onehot = (iota == code).astype(jnp.float32).astype(jnp.bfloat16)
    dn = (((1,), (0,)), ((), ()))
    delta = (
        jax.lax.dot_general(onehot, hi, dn, preferred_element_type=jnp.float32)
        + jax.lax.dot_general(onehot, mid, dn, preferred_element_type=jnp.float32)
        + jax.lax.dot_general(onehot, lo, dn, preferred_element_type=jnp.float32)
    )
    return resid - delta, code


def _body(x_ref, cb_ref, cbn_ref, hi_ref, mid_ref, lo_ref, out_ref):
    resids = [x_ref[0:HALF], x_ref[HALF:TILE]]
    codes = [[], []]
    for l in range(N_LAYERS):
        cb, cbn = cb_ref[l], cbn_ref[l][None, :]
        hi, mid, lo = hi_ref[l], mid_ref[l], lo_ref[l]
        for s in range(2):
            resids[s], code = _layer(resids[s], cb, cbn, hi, mid, lo)
            codes[s].append(code)
    out_ref[0:HALF, :] = jnp.concatenate(codes[0], axis=1)
    out_ref[HALF:TILE, :] = jnp.concatenate(codes[1], axis=1)


@jax.jit
def kernel(x, centroids):
    n = x.shape[0]
    cb_norm = jnp.sum(centroids * centroids, axis=2)
    hi = centroids.astype(jnp.bfloat16)
    r1 = centroids - hi.astype(jnp.float32)
    mid = r1.astype(jnp.bfloat16)
    lo = (r1 - mid.astype(jnp.float32)).astype(jnp.bfloat16)
    full = lambda s: pl.BlockSpec(s, lambda i: (0,) * len(s))
    return pl.pallas_call(
        _body,
        grid=(n // TILE,),
        in_specs=[
            pl.BlockSpec((TILE, DIM), lambda i: (i, 0)),
            full((N_LAYERS, K, DIM)),
            full((N_LAYERS, K)),
            full((N_LAYERS, K, DIM)),
            full((N_LAYERS, K, DIM)),
            full((N_LAYERS, K, DIM)),
        ],
        out_specs=pl.BlockSpec((TILE, N_LAYERS), lambda i: (i, 0)),
        out_shape=jax.ShapeDtypeStruct((n, N_LAYERS), jnp.int32),
    )(x, centroids, cb_norm, hi, mid, lo)


# speedup vs baseline: 3.2497x; 1.7767x over previous
import functools

import jax
import jax.numpy as jnp
from jax.experimental import pallas as pl

N_LAYERS = 4
K = 1024
DIM = 64
HALF = 256
NSUB = 8
TILE = NSUB * HALF


def _layer(resid, cb, cb_norm, cbsplit):
    x_norm = jnp.sum(resid * resid, axis=1, keepdims=True)
    mm2 = jax.lax.dot_general(
        -2.0 * resid, cb, (((1,), (1,)), ((), ())),
        preferred_element_type=jnp.float32,
    )
    d = (x_norm + cb_norm) + mm2
    d_min = jnp.min(d, axis=1, keepdims=True)
    iota = jax.lax.broadcasted_iota(jnp.int32, d.shape, 1)
    code = jnp.min(jnp.where(d == d_min, iota, K), axis=1, keepdims=True)
    onehot = (iota == code).astype(jnp.float32).astype(jnp.bfloat16)
    dn = (((1,), (0,)), ((), ()))
    d3 = jax.lax.dot_general(onehot, cbsplit, dn,
                             preferred_element_type=jnp.float32)
    delta = (d3[:, :DIM] + d3[:, DIM:2 * DIM]) + d3[:, 2 * DIM:]
    return resid - delta, code


def _body(x_ref, cb_ref, cbn_ref, cbsplit_ref, out_ref):
    resids = [x_ref[s * HALF:(s + 1) * HALF] for s in range(NSUB)]
    codes = [[] for _ in range(NSUB)]
    for l in range(N_LAYERS):
        cb, cbn, cbsplit = cb_ref[l], cbn_ref[l][None, :], cbsplit_ref[l]
        for s in range(NSUB):
            resids[s], code = _layer(resids[s], cb, cbn, cbsplit)
            codes[s].append(code)
    for s in range(NSUB):
        out_ref[s * HALF:(s + 1) * HALF, :] = jnp.concatenate(codes[s], axis=1)


@jax.jit
def kernel(x, centroids):
    n = x.shape[0]
    cb_norm = jnp.sum(centroids * centroids, axis=2)
    hi = centroids.astype(jnp.bfloat16)
    r1 = centroids - hi.astype(jnp.float32)
    mid = r1.astype(jnp.bfloat16)
    lo = (r1 - mid.astype(jnp.float32)).astype(jnp.bfloat16)
    cbsplit = jnp.concatenate([hi, mid, lo], axis=2)
    full = lambda s: pl.BlockSpec(s, lambda i: (0,) * len(s))
    return pl.pallas_call(
        _body,
        grid=(n // TILE,),
        in_specs=[
            pl.BlockSpec((TILE, DIM), lambda i: (i, 0)),
            full((N_LAYERS, K, DIM)),
            full((N_LAYERS, K)),
            full((N_LAYERS, K, 3 * DIM)),
        ],
        out_specs=pl.BlockSpec((TILE, N_LAYERS), lambda i: (i, 0)),
        out_shape=jax.ShapeDtypeStruct((n, N_LAYERS), jnp.int32),
    )(x, centroids, cb_norm, cbsplit)
